# initial kernel scaffold (unmeasured)
import jax
import jax.numpy as jnp
from jax import lax
from jax.experimental import pallas as pl
from jax.experimental.pallas import tpu as pltpu

N_DEV = 4
M_PER = 1024
K = 4096
N_TOT = 8192
N_PER = 2048
WBLK = 256
G = N_TOT // WBLK


def _mesh_id(peer):
    return (peer,)


def _compute_quant_a2a(x, w_mat):

    def body(x_ref, w_ref, recv_ref, scale_ref, y_ref, mymax_ref, maxes_ref,
             send_buf, max_send_sems, max_recv_sems, data_send_sems,
             data_recv_sems):
        j = pl.program_id(0)
        my = lax.axis_index("i")

        yblk = jnp.dot(x_ref[...], w_ref[...],
                       preferred_element_type=jnp.float32)
        y_ref[:, pl.ds(j * WBLK, WBLK)] = yblk.astype(jnp.bfloat16)
        m = jnp.maximum(jnp.max(yblk), 0.0)
        prev = jnp.where(j == 0, 0.0, mymax_ref[0, 0])
        mymax_ref[...] = jnp.full((8, 128), jnp.maximum(prev, m),
                                  dtype=jnp.float32)

        @pl.when(j == G - 1)
        def _last_step():
            bsem = pltpu.get_barrier_semaphore()
            for d in range(1, N_DEV):
                peer = lax.rem(my + d, N_DEV)
                pl.semaphore_signal(bsem, inc=1, device_id=_mesh_id(peer),
                                    device_id_type=pl.DeviceIdType.MESH)
            pl.semaphore_wait(bsem, N_DEV - 1)

            max_sends = []
            for d in range(1, N_DEV):
                peer = lax.rem(my + d, N_DEV)
                rdma = pltpu.make_async_remote_copy(
                    src_ref=mymax_ref,
                    dst_ref=maxes_ref.at[d - 1],
                    send_sem=max_send_sems.at[d - 1],
                    recv_sem=max_recv_sems.at[d - 1],
                    device_id=_mesh_id(peer),
                    device_id_type=pl.DeviceIdType.MESH,
                )
                rdma.start()
                max_sends.append(rdma)
            for e in range(N_DEV - 1):
                rx = pltpu.make_async_remote_copy(
                    src_ref=mymax_ref,
                    dst_ref=maxes_ref.at[e],
                    send_sem=max_send_sems.at[e],
                    recv_sem=max_recv_sems.at[e],
                    device_id=_mesh_id(my),
                    device_id_type=pl.DeviceIdType.MESH,
                )
                rx.wait_recv()

            gmax = jnp.maximum(jnp.max(maxes_ref[...]), mymax_ref[0, 0])
            scale = gmax / 127.0
            scale_ref[...] = jnp.full((8, 128), scale, dtype=jnp.float32)
            inv = 127.0 / gmax

            def quant(col_start):
                yv = y_ref[:, pl.ds(col_start, N_PER)].astype(jnp.float32)
                yv = jnp.maximum(yv, 0.0)
                q = jnp.clip(jnp.round(yv * inv), 0.0, 127.0)
                return q.astype(jnp.int8)

            recv_ref[3] = quant(my * N_PER)

            data_sends = []
            for d in range(1, N_DEV):
                peer = lax.rem(my + d, N_DEV)
                send_buf[d - 1] = quant(peer * N_PER)
                rdma = pltpu.make_async_remote_copy(
                    src_ref=send_buf.at[d - 1],
                    dst_ref=recv_ref.at[d - 1],
                    send_sem=data_send_sems.at[d - 1],
                    recv_sem=data_recv_sems.at[d - 1],
                    device_id=_mesh_id(peer),
                    device_id_type=pl.DeviceIdType.MESH,
                )
                rdma.start()
                data_sends.append(rdma)

            for e in range(N_DEV - 1):
                rx = pltpu.make_async_remote_copy(
                    src_ref=send_buf.at[e],
                    dst_ref=recv_ref.at[e],
                    send_sem=data_send_sems.at[e],
                    recv_sem=data_recv_sems.at[e],
                    device_id=_mesh_id(my),
                    device_id_type=pl.DeviceIdType.MESH,
                )
                rx.wait_recv()
            for s in max_sends + data_sends:
                s.wait_send()

    return pl.pallas_call(
        body,
        grid=(G,),
        out_shape=(
            jax.ShapeDtypeStruct((N_DEV, M_PER, N_PER), jnp.int8),
            jax.ShapeDtypeStruct((8, 128), jnp.float32),
        ),
        in_specs=[
            pl.BlockSpec((M_PER, K), lambda j: (0, 0)),
            pl.BlockSpec((K, WBLK), lambda j: (0, j)),
        ],
        out_specs=(
            pl.BlockSpec((N_DEV, M_PER, N_PER), lambda j: (0, 0, 0)),
            pl.BlockSpec((8, 128), lambda j: (0, 0)),
        ),
        scratch_shapes=[
            pltpu.VMEM((M_PER, N_TOT), jnp.bfloat16),
            pltpu.VMEM((8, 128), jnp.float32),
            pltpu.VMEM((N_DEV - 1, 8, 128), jnp.float32),
            pltpu.VMEM((N_DEV - 1, M_PER, N_PER), jnp.int8),
            pltpu.SemaphoreType.DMA((N_DEV - 1,)),
            pltpu.SemaphoreType.DMA((N_DEV - 1,)),
            pltpu.SemaphoreType.DMA((N_DEV - 1,)),
            pltpu.SemaphoreType.DMA((N_DEV - 1,)),
        ],
        compiler_params=pltpu.CompilerParams(
            dimension_semantics=("arbitrary",),
            collective_id=0,
        ),
    )(x, w_mat)


def _dequant(recv_q, scale):

    def body(recv_ref, scale_ref, out_ref):
        my = lax.axis_index("i")
        s = scale_ref[0, 0]
        for b in range(N_DEV):
            k = jnp.where(b == my, 3, lax.rem(my - b + N_DEV, N_DEV) - 1)
            chunk = recv_ref[pl.ds(k, 1)]
            out_ref[pl.ds(b * M_PER, M_PER), :] = (
                chunk[0].astype(jnp.float32) * s)

    return pl.pallas_call(
        body,
        out_shape=jax.ShapeDtypeStruct((N_DEV * M_PER, N_PER), jnp.float32),
        in_specs=[
            pl.BlockSpec(memory_space=pltpu.VMEM),
            pl.BlockSpec(memory_space=pltpu.VMEM),
        ],
        out_specs=pl.BlockSpec(memory_space=pltpu.VMEM),
    )(recv_q, scale)


def kernel(x, w_mat):
    recv_q, scale = _compute_quant_a2a(x, w_mat)
    return _dequant(recv_q, scale)


# baseline (device time: 253875 ns/iter reference)
import jax
import jax.numpy as jnp
from jax import lax
from jax.experimental import pallas as pl
from jax.experimental.pallas import tpu as pltpu

N_DEV = 4
M_PER = 1024
K = 4096
N_TOT = 8192
N_PER = 2048
WBLK = 128
G = N_TOT // WBLK


def _mesh_id(peer):
    return (peer,)


def _compute_quant_a2a(x, w_mat):

    def body(x_ref, w_ref, recv_ref, scale_ref, y_ref, mymax_ref, maxes_ref,
             send_buf, max_send_sems, max_recv_sems, data_send_sems,
             data_recv_sems):
        j = pl.program_id(0)
        my = lax.axis_index("i")

        yblk = jnp.dot(x_ref[...], w_ref[...],
                       preferred_element_type=jnp.float32)
        y_ref[:, pl.ds(j * WBLK, WBLK)] = yblk.astype(jnp.bfloat16)
        m = jnp.maximum(jnp.max(yblk), 0.0)
        prev = jnp.where(j == 0, 0.0, mymax_ref[0, 0])
        mymax_ref[...] = jnp.full((8, 128), jnp.maximum(prev, m),
                                  dtype=jnp.float32)

        @pl.when(j == G - 1)
        def _last_step():
            bsem = pltpu.get_barrier_semaphore()
            for d in range(1, N_DEV):
                peer = lax.rem(my + d, N_DEV)
                pl.semaphore_signal(bsem, inc=1, device_id=_mesh_id(peer),
                                    device_id_type=pl.DeviceIdType.MESH)
            pl.semaphore_wait(bsem, N_DEV - 1)

            max_sends = []
            for d in range(1, N_DEV):
                peer = lax.rem(my + d, N_DEV)
                rdma = pltpu.make_async_remote_copy(
                    src_ref=mymax_ref,
                    dst_ref=maxes_ref.at[d - 1],
                    send_sem=max_send_sems.at[d - 1],
                    recv_sem=max_recv_sems.at[d - 1],
                    device_id=_mesh_id(peer),
                    device_id_type=pl.DeviceIdType.MESH,
                )
                rdma.start()
                max_sends.append(rdma)
            for e in range(N_DEV - 1):
                rx = pltpu.make_async_remote_copy(
                    src_ref=mymax_ref,
                    dst_ref=maxes_ref.at[e],
                    send_sem=max_send_sems.at[e],
                    recv_sem=max_recv_sems.at[e],
                    device_id=_mesh_id(my),
                    device_id_type=pl.DeviceIdType.MESH,
                )
                rx.wait_recv()

            gmax = jnp.maximum(jnp.max(maxes_ref[...]), mymax_ref[0, 0])
            scale = gmax / 127.0
            scale_ref[...] = jnp.full((8, 128), scale, dtype=jnp.float32)
            inv = 127.0 / gmax

            def quant(col_start):
                yv = y_ref[:, pl.ds(col_start, N_PER)].astype(jnp.float32)
                yv = jnp.maximum(yv, 0.0)
                q = jnp.clip(jnp.round(yv * inv), 0.0, 127.0)
                return q.astype(jnp.int8)

            recv_ref[3] = quant(my * N_PER)

            data_sends = []
            for d in range(1, N_DEV):
                peer = lax.rem(my + d, N_DEV)
                send_buf[d - 1] = quant(peer * N_PER)
                rdma = pltpu.make_async_remote_copy(
                    src_ref=send_buf.at[d - 1],
                    dst_ref=recv_ref.at[d - 1],
                    send_sem=data_send_sems.at[d - 1],
                    recv_sem=data_recv_sems.at[d - 1],
                    device_id=_mesh_id(peer),
                    device_id_type=pl.DeviceIdType.MESH,
                )
                rdma.start()
                data_sends.append(rdma)

            for e in range(N_DEV - 1):
                rx = pltpu.make_async_remote_copy(
                    src_ref=send_buf.at[e],
                    dst_ref=recv_ref.at[e],
                    send_sem=data_send_sems.at[e],
                    recv_sem=data_recv_sems.at[e],
                    device_id=_mesh_id(my),
                    device_id_type=pl.DeviceIdType.MESH,
                )
                rx.wait_recv()
            for s in max_sends + data_sends:
                s.wait_send()

    return pl.pallas_call(
        body,
        grid=(G,),
        out_shape=(
            jax.ShapeDtypeStruct((N_DEV, M_PER, N_PER), jnp.int8),
            jax.ShapeDtypeStruct((8, 128), jnp.float32),
        ),
        in_specs=[
            pl.BlockSpec((M_PER, K), lambda j: (0, 0)),
            pl.BlockSpec((K, WBLK), lambda j: (0, j)),
        ],
        out_specs=(
            pl.BlockSpec((N_DEV, M_PER, N_PER), lambda j: (0, 0, 0)),
            pl.BlockSpec((8, 128), lambda j: (0, 0)),
        ),
        scratch_shapes=[
            pltpu.VMEM((M_PER, N_TOT), jnp.bfloat16),
            pltpu.VMEM((8, 128), jnp.float32),
            pltpu.VMEM((N_DEV - 1, 8, 128), jnp.float32),
            pltpu.VMEM((N_DEV - 1, M_PER, N_PER), jnp.int8),
            pltpu.SemaphoreType.DMA((N_DEV - 1,)),
            pltpu.SemaphoreType.DMA((N_DEV - 1,)),
            pltpu.SemaphoreType.DMA((N_DEV - 1,)),
            pltpu.SemaphoreType.DMA((N_DEV - 1,)),
        ],
        compiler_params=pltpu.CompilerParams(
            dimension_semantics=("arbitrary",),
            collective_id=0,
            vmem_limit_bytes=64 * 1024 * 1024,
        ),
    )(x, w_mat)


def _dequant(recv_q, scale):

    def body(recv_ref, scale_ref, out_ref):
        b = pl.program_id(0)
        my = lax.axis_index("i")
        s = scale_ref[0, 0]
        k = jnp.where(b == my, 3, lax.rem(my - b + N_DEV, N_DEV) - 1)
        chunk = recv_ref[pl.ds(k, 1)]
        out_ref[...] = chunk[0].astype(jnp.float32) * s

    return pl.pallas_call(
        body,
        grid=(N_DEV,),
        out_shape=jax.ShapeDtypeStruct((N_DEV * M_PER, N_PER), jnp.float32),
        in_specs=[
            pl.BlockSpec((N_DEV, M_PER, N_PER), lambda b: (0, 0, 0)),
            pl.BlockSpec((8, 128), lambda b: (0, 0)),
        ],
        out_specs=pl.BlockSpec((M_PER, N_PER), lambda b: (b, 0)),
        compiler_params=pltpu.CompilerParams(
            dimension_semantics=("arbitrary",),
        ),
    )(recv_q, scale)


def kernel(x, w_mat):
    recv_q, scale = _compute_quant_a2a(x, w_mat)
    return _dequant(recv_q, scale)


# device time: 174623 ns/iter; 1.4538x vs baseline; 1.4538x over previous
import jax
import jax.numpy as jnp
from jax import lax
from jax.experimental import pallas as pl
from jax.experimental.pallas import tpu as pltpu

N_DEV = 4
M_PER = 1024
K = 4096
N_TOT = 8192
N_PER = 2048
WBLK = 512
G = N_TOT // WBLK


def _mesh_id(peer):
    return (peer,)


def _cast_bf16(x):

    def body(x_ref, o_ref):
        o_ref[...] = x_ref[...].astype(jnp.bfloat16)

    return pl.pallas_call(
        body,
        grid=(4,),
        out_shape=jax.ShapeDtypeStruct((M_PER, K), jnp.bfloat16),
        in_specs=[pl.BlockSpec((M_PER // 4, K), lambda j: (j, 0))],
        out_specs=pl.BlockSpec((M_PER // 4, K), lambda j: (j, 0)),
        compiler_params=pltpu.CompilerParams(
            dimension_semantics=("arbitrary",),
        ),
    )(x)


def _compute_quant_a2a(x, w_mat):

    def body(x_ref, w_ref, recv_ref, scale_ref, y_ref, mymax_ref, maxes_ref,
             send_buf, max_send_sems, max_recv_sems, data_send_sems,
             data_recv_sems):
        j = pl.program_id(0)
        my = lax.axis_index("i")

        yblk = jnp.dot(x_ref[...], w_ref[...].astype(jnp.bfloat16),
                       preferred_element_type=jnp.float32)
        y_ref[:, pl.ds(j * WBLK, WBLK)] = yblk.astype(jnp.bfloat16)
        m = jnp.maximum(jnp.max(yblk), 0.0)
        prev = jnp.where(j == 0, 0.0, mymax_ref[0, 0])
        mymax_ref[...] = jnp.full((8, 128), jnp.maximum(prev, m),
                                  dtype=jnp.float32)

        @pl.when(j == G - 1)
        def _last_step():
            bsem = pltpu.get_barrier_semaphore()
            for d in range(1, N_DEV):
                peer = lax.rem(my + d, N_DEV)
                pl.semaphore_signal(bsem, inc=1, device_id=_mesh_id(peer),
                                    device_id_type=pl.DeviceIdType.MESH)
            pl.semaphore_wait(bsem, N_DEV - 1)

            max_sends = []
            for d in range(1, N_DEV):
                peer = lax.rem(my + d, N_DEV)
                rdma = pltpu.make_async_remote_copy(
                    src_ref=mymax_ref,
                    dst_ref=maxes_ref.at[d - 1],
                    send_sem=max_send_sems.at[d - 1],
                    recv_sem=max_recv_sems.at[d - 1],
                    device_id=_mesh_id(peer),
                    device_id_type=pl.DeviceIdType.MESH,
                )
                rdma.start()
                max_sends.append(rdma)
            for e in range(N_DEV - 1):
                rx = pltpu.make_async_remote_copy(
                    src_ref=mymax_ref,
                    dst_ref=maxes_ref.at[e],
                    send_sem=max_send_sems.at[e],
                    recv_sem=max_recv_sems.at[e],
                    device_id=_mesh_id(my),
                    device_id_type=pl.DeviceIdType.MESH,
                )
                rx.wait_recv()

            gmax = jnp.maximum(jnp.max(maxes_ref[...]), mymax_ref[0, 0])
            scale = gmax / 127.0
            scale_ref[...] = jnp.full((8, 128), scale, dtype=jnp.float32)
            inv = 127.0 / gmax

            def quant(col_start):
                yv = y_ref[:, pl.ds(col_start, N_PER)].astype(jnp.float32)
                yv = jnp.maximum(yv, 0.0)
                q = jnp.clip(jnp.round(yv * inv), 0.0, 127.0)
                return q.astype(jnp.int8)

            recv_ref[3] = quant(my * N_PER)

            data_sends = []
            for d in range(1, N_DEV):
                peer = lax.rem(my + d, N_DEV)
                send_buf[d - 1] = quant(peer * N_PER)
                rdma = pltpu.make_async_remote_copy(
                    src_ref=send_buf.at[d - 1],
                    dst_ref=recv_ref.at[d - 1],
                    send_sem=data_send_sems.at[d - 1],
                    recv_sem=data_recv_sems.at[d - 1],
                    device_id=_mesh_id(peer),
                    device_id_type=pl.DeviceIdType.MESH,
                )
                rdma.start()
                data_sends.append(rdma)

            for e in range(N_DEV - 1):
                rx = pltpu.make_async_remote_copy(
                    src_ref=send_buf.at[e],
                    dst_ref=recv_ref.at[e],
                    send_sem=data_send_sems.at[e],
                    recv_sem=data_recv_sems.at[e],
                    device_id=_mesh_id(my),
                    device_id_type=pl.DeviceIdType.MESH,
                )
                rx.wait_recv()
            for s in max_sends + data_sends:
                s.wait_send()

    return pl.pallas_call(
        body,
        grid=(G,),
        out_shape=(
            jax.ShapeDtypeStruct((N_DEV, M_PER, N_PER), jnp.int8),
            jax.ShapeDtypeStruct((8, 128), jnp.float32),
        ),
        in_specs=[
            pl.BlockSpec((M_PER, K), lambda j: (0, 0)),
            pl.BlockSpec((K, WBLK), lambda j: (0, j)),
        ],
        out_specs=(
            pl.BlockSpec((N_DEV, M_PER, N_PER), lambda j: (0, 0, 0)),
            pl.BlockSpec((8, 128), lambda j: (0, 0)),
        ),
        scratch_shapes=[
            pltpu.VMEM((M_PER, N_TOT), jnp.bfloat16),
            pltpu.VMEM((8, 128), jnp.float32),
            pltpu.VMEM((N_DEV - 1, 8, 128), jnp.float32),
            pltpu.VMEM((N_DEV - 1, M_PER, N_PER), jnp.int8),
            pltpu.SemaphoreType.DMA((N_DEV - 1,)),
            pltpu.SemaphoreType.DMA((N_DEV - 1,)),
            pltpu.SemaphoreType.DMA((N_DEV - 1,)),
            pltpu.SemaphoreType.DMA((N_DEV - 1,)),
        ],
        compiler_params=pltpu.CompilerParams(
            dimension_semantics=("arbitrary",),
            collective_id=0,
            vmem_limit_bytes=64 * 1024 * 1024,
        ),
    )(x, w_mat)


def _dequant(recv_q, scale):

    def body(recv_ref, scale_ref, out_ref):
        b = pl.program_id(0)
        my = lax.axis_index("i")
        s = scale_ref[0, 0]
        k = jnp.where(b == my, 3, lax.rem(my - b + N_DEV, N_DEV) - 1)
        chunk = recv_ref[pl.ds(k, 1)]
        out_ref[...] = chunk[0].astype(jnp.float32) * s

    return pl.pallas_call(
        body,
        grid=(N_DEV,),
        out_shape=jax.ShapeDtypeStruct((N_DEV * M_PER, N_PER), jnp.float32),
        in_specs=[
            pl.BlockSpec((N_DEV, M_PER, N_PER), lambda b: (0, 0, 0)),
            pl.BlockSpec((8, 128), lambda b: (0, 0)),
        ],
        out_specs=pl.BlockSpec((M_PER, N_PER), lambda b: (b, 0)),
        compiler_params=pltpu.CompilerParams(
            dimension_semantics=("arbitrary",),
        ),
    )(recv_q, scale)


def kernel(x, w_mat):
    recv_q, scale = _compute_quant_a2a(_cast_bf16(x), w_mat)
    return _dequant(recv_q, scale)


# device time: 155607 ns/iter; 1.6315x vs baseline; 1.1222x over previous
import jax
import jax.numpy as jnp
from jax import lax
from jax.experimental import pallas as pl
from jax.experimental.pallas import tpu as pltpu

N_DEV = 4
M_PER = 1024
K = 4096
N_TOT = 8192
N_PER = 2048
WBLK = 256
G = N_TOT // WBLK
STEPS_PER_CHUNK = N_PER // WBLK


def _mesh_id(peer):
    return (peer,)


_CHUNK_OFF_LIST = (1, 3, 2, 0)


def _CHUNK_OFF(k):
    return jnp.where(k == 0, 1, jnp.where(k == 1, 3, jnp.where(k == 2, 2, 0)))


def _w_index_map(j):
    my = lax.axis_index("i")
    k = j // STEPS_PER_CHUNK
    chunk_c = lax.rem(my + _CHUNK_OFF(k), N_DEV)
    return 0, chunk_c * STEPS_PER_CHUNK + lax.rem(j, STEPS_PER_CHUNK)


def _cast_bf16(x):

    def body(x_ref, o_ref):
        o_ref[...] = x_ref[...].astype(jnp.bfloat16)

    return pl.pallas_call(
        body,
        grid=(4,),
        out_shape=jax.ShapeDtypeStruct((M_PER, K), jnp.bfloat16),
        in_specs=[pl.BlockSpec((M_PER // 4, K), lambda j: (j, 0))],
        out_specs=pl.BlockSpec((M_PER // 4, K), lambda j: (j, 0)),
        compiler_params=pltpu.CompilerParams(
            dimension_semantics=("arbitrary",),
        ),
    )(x)


def _compute_quant_a2a(x, w_mat):

    def body(x_ref, w_ref, recv_ref, scale_ref, y_ref, mymax_ref, maxes_ref,
             send_buf, bf_recv, max_send_sems, max_recv_sems, diag_send_sem,
             diag_recv_sem, bf_send_sems, bf_recv_sems):
        j = pl.program_id(0)
        my = lax.axis_index("i")
        k = j // STEPS_PER_CHUNK
        chunk_c = lax.rem(my + _CHUNK_OFF(k), N_DEV)
        col_blk = chunk_c * STEPS_PER_CHUNK + lax.rem(j, STEPS_PER_CHUNK)

        @pl.when(j == 0)
        def _barrier():
            bsem = pltpu.get_barrier_semaphore()
            for d in range(1, N_DEV):
                peer = lax.rem(my + d, N_DEV)
                pl.semaphore_signal(bsem, inc=1, device_id=_mesh_id(peer),
                                    device_id_type=pl.DeviceIdType.MESH)
            pl.semaphore_wait(bsem, N_DEV - 1)

        yblk = jnp.dot(x_ref[...], w_ref[...].astype(jnp.bfloat16),
                       preferred_element_type=jnp.float32)
        y_ref[:, pl.ds(col_blk * WBLK, WBLK)] = yblk.astype(jnp.bfloat16)
        m = jnp.maximum(jnp.max(yblk), 0.0)
        prev = jnp.where(j == 0, 0.0, mymax_ref[0, 0])
        mymax_ref[...] = jnp.full((8, 128), jnp.maximum(prev, m),
                                  dtype=jnp.float32)

        for kk, (off, idx) in enumerate(((1, 0), (3, 1))):
            @pl.when(j == (kk + 1) * STEPS_PER_CHUNK - 1)
            def _eager(off=off, idx=idx):
                peer = lax.rem(my + off, N_DEV)
                rdma = pltpu.make_async_remote_copy(
                    src_ref=y_ref.at[:, pl.ds(peer * N_PER, N_PER)],
                    dst_ref=bf_recv.at[idx],
                    send_sem=bf_send_sems.at[idx],
                    recv_sem=bf_recv_sems.at[idx],
                    device_id=_mesh_id(peer),
                    device_id_type=pl.DeviceIdType.MESH,
                )
                rdma.start()

        @pl.when(j == G - 1)
        def _last_step():
            max_sends = []
            for d in range(1, N_DEV):
                peer = lax.rem(my + d, N_DEV)
                rdma = pltpu.make_async_remote_copy(
                    src_ref=mymax_ref,
                    dst_ref=maxes_ref.at[d - 1],
                    send_sem=max_send_sems.at[d - 1],
                    recv_sem=max_recv_sems.at[d - 1],
                    device_id=_mesh_id(peer),
                    device_id_type=pl.DeviceIdType.MESH,
                )
                rdma.start()
                max_sends.append(rdma)
            for e in range(N_DEV - 1):
                rx = pltpu.make_async_remote_copy(
                    src_ref=mymax_ref,
                    dst_ref=maxes_ref.at[e],
                    send_sem=max_send_sems.at[e],
                    recv_sem=max_recv_sems.at[e],
                    device_id=_mesh_id(my),
                    device_id_type=pl.DeviceIdType.MESH,
                )
                rx.wait_recv()

            gmax = jnp.maximum(jnp.max(maxes_ref[...]), mymax_ref[0, 0])
            scale = gmax / 127.0
            scale_ref[...] = jnp.full((8, 128), scale, dtype=jnp.float32)
            inv = 127.0 / gmax

            def quant_f32(yv):
                yv = jnp.maximum(yv, 0.0)
                q = jnp.clip(jnp.round(yv * inv), 0.0, 127.0)
                return q.astype(jnp.int8)

            diag = lax.rem(my + 2, N_DEV)
            send_buf[...] = quant_f32(
                y_ref[:, pl.ds(diag * N_PER, N_PER)].astype(jnp.float32))
            diag_rdma = pltpu.make_async_remote_copy(
                src_ref=send_buf,
                dst_ref=recv_ref.at[1],
                send_sem=diag_send_sem,
                recv_sem=diag_recv_sem,
                device_id=_mesh_id(diag),
                device_id_type=pl.DeviceIdType.MESH,
            )
            diag_rdma.start()

            recv_ref[3] = quant_f32(
                y_ref[:, pl.ds(my * N_PER, N_PER)].astype(jnp.float32))

            for idx, slot in ((0, 0), (1, 2)):
                rx = pltpu.make_async_remote_copy(
                    src_ref=y_ref.at[:, pl.ds(0, N_PER)],
                    dst_ref=bf_recv.at[idx],
                    send_sem=bf_send_sems.at[idx],
                    recv_sem=bf_recv_sems.at[idx],
                    device_id=_mesh_id(my),
                    device_id_type=pl.DeviceIdType.MESH,
                )
                rx.wait_recv()
                recv_ref[slot] = quant_f32(bf_recv[idx].astype(jnp.float32))

            rx = pltpu.make_async_remote_copy(
                src_ref=send_buf,
                dst_ref=recv_ref.at[1],
                send_sem=diag_send_sem,
                recv_sem=diag_recv_sem,
                device_id=_mesh_id(my),
                device_id_type=pl.DeviceIdType.MESH,
            )
            rx.wait_recv()

            for s in max_sends:
                s.wait_send()
            diag_rdma.wait_send()
            for idx in range(2):
                tx = pltpu.make_async_remote_copy(
                    src_ref=y_ref.at[:, pl.ds(0, N_PER)],
                    dst_ref=bf_recv.at[idx],
                    send_sem=bf_send_sems.at[idx],
                    recv_sem=bf_recv_sems.at[idx],
                    device_id=_mesh_id(my),
                    device_id_type=pl.DeviceIdType.MESH,
                )
                tx.wait_send()

    return pl.pallas_call(
        body,
        grid=(G,),
        out_shape=(
            jax.ShapeDtypeStruct((N_DEV, M_PER, N_PER), jnp.int8),
            jax.ShapeDtypeStruct((8, 128), jnp.float32),
        ),
        in_specs=[
            pl.BlockSpec((M_PER, K), lambda j: (0, 0)),
            pl.BlockSpec((K, WBLK), _w_index_map),
        ],
        out_specs=(
            pl.BlockSpec((N_DEV, M_PER, N_PER), lambda j: (0, 0, 0)),
            pl.BlockSpec((8, 128), lambda j: (0, 0)),
        ),
        scratch_shapes=[
            pltpu.VMEM((M_PER, N_TOT), jnp.bfloat16),
            pltpu.VMEM((8, 128), jnp.float32),
            pltpu.VMEM((N_DEV - 1, 8, 128), jnp.float32),
            pltpu.VMEM((M_PER, N_PER), jnp.int8),
            pltpu.VMEM((2, M_PER, N_PER), jnp.bfloat16),
            pltpu.SemaphoreType.DMA((N_DEV - 1,)),
            pltpu.SemaphoreType.DMA((N_DEV - 1,)),
            pltpu.SemaphoreType.DMA,
            pltpu.SemaphoreType.DMA,
            pltpu.SemaphoreType.DMA((2,)),
            pltpu.SemaphoreType.DMA((2,)),
        ],
        compiler_params=pltpu.CompilerParams(
            dimension_semantics=("arbitrary",),
            collective_id=0,
            vmem_limit_bytes=64 * 1024 * 1024,
            skip_device_barrier=True,
        ),
    )(x, w_mat)


def _dequant(recv_q, scale):

    def body(recv_ref, scale_ref, out_ref):
        b = pl.program_id(0)
        my = lax.axis_index("i")
        s = scale_ref[0, 0]
        k = jnp.where(b == my, 3, lax.rem(my - b + N_DEV, N_DEV) - 1)
        chunk = recv_ref[pl.ds(k, 1)]
        out_ref[...] = chunk[0].astype(jnp.float32) * s

    return pl.pallas_call(
        body,
        grid=(N_DEV,),
        out_shape=jax.ShapeDtypeStruct((N_DEV * M_PER, N_PER), jnp.float32),
        in_specs=[
            pl.BlockSpec((N_DEV, M_PER, N_PER), lambda b: (0, 0, 0)),
            pl.BlockSpec((8, 128), lambda b: (0, 0)),
        ],
        out_specs=pl.BlockSpec((M_PER, N_PER), lambda b: (b, 0)),
        compiler_params=pltpu.CompilerParams(
            dimension_semantics=("arbitrary",),
        ),
    )(recv_q, scale)


def kernel(x, w_mat):
    recv_q, scale = _compute_quant_a2a(_cast_bf16(x), w_mat)
    return _dequant(recv_q, scale)


# device time: 152732 ns/iter; 1.6622x vs baseline; 1.0188x over previous
import jax
import jax.numpy as jnp
from jax import lax
from jax.experimental import pallas as pl
from jax.experimental.pallas import tpu as pltpu

N_DEV = 4
M_PER = 1024
K = 4096
N_TOT = 8192
N_PER = 2048
WBLK = 256
G = N_TOT // WBLK
STEPS_PER_CHUNK = N_PER // WBLK


def _mesh_id(peer):
    return (peer,)


_CHUNK_OFF_LIST = (1, 3, 2, 0)


def _CHUNK_OFF(k):
    return jnp.where(k == 0, 1, jnp.where(k == 1, 3, jnp.where(k == 2, 2, 0)))


def _w_index_map(j):
    my = lax.axis_index("i")
    k = j // STEPS_PER_CHUNK
    chunk_c = lax.rem(my + _CHUNK_OFF(k), N_DEV)
    return 0, chunk_c * STEPS_PER_CHUNK + lax.rem(j, STEPS_PER_CHUNK)


def _cast_bf16(x):

    def body(x_ref, o_ref):
        o_ref[...] = x_ref[...].astype(jnp.bfloat16)

    return pl.pallas_call(
        body,
        grid=(4,),
        out_shape=jax.ShapeDtypeStruct((M_PER, K), jnp.bfloat16),
        in_specs=[pl.BlockSpec((M_PER // 4, K), lambda j: (j, 0))],
        out_specs=pl.BlockSpec((M_PER // 4, K), lambda j: (j, 0)),
        compiler_params=pltpu.CompilerParams(
            dimension_semantics=("arbitrary",),
        ),
    )(x)


def _compute_quant_a2a(x, w_mat):

    def body(x_ref, w_ref, recv_ref, scale_ref, y_ref, mymax_ref, maxes_ref,
             send_buf, bf_recv, max_send_sems, max_recv_sems, diag_send_sem,
             diag_recv_sem, bf_send_sems, bf_recv_sems):
        j = pl.program_id(0)
        my = lax.axis_index("i")
        k = j // STEPS_PER_CHUNK
        chunk_c = lax.rem(my + _CHUNK_OFF(k), N_DEV)
        col_blk = chunk_c * STEPS_PER_CHUNK + lax.rem(j, STEPS_PER_CHUNK)

        @pl.when(j == 0)
        def _barrier():
            with jax.named_scope("entry_barrier"):
                bsem = pltpu.get_barrier_semaphore()
                for d in range(1, N_DEV):
                    peer = lax.rem(my + d, N_DEV)
                    pl.semaphore_signal(bsem, inc=1, device_id=_mesh_id(peer),
                                        device_id_type=pl.DeviceIdType.MESH)
                pl.semaphore_wait(bsem, N_DEV - 1)

        yblk = jnp.dot(x_ref[...], w_ref[...].astype(jnp.bfloat16),
                       preferred_element_type=jnp.float32)
        y_ref[:, pl.ds(col_blk * WBLK, WBLK)] = yblk.astype(jnp.bfloat16)
        m = jnp.maximum(jnp.max(yblk), 0.0)
        prev = jnp.where(j == 0, 0.0, mymax_ref[0, 0])
        mymax_ref[...] = jnp.full((8, 128), jnp.maximum(prev, m),
                                  dtype=jnp.float32)

        for kk, (off, idx) in enumerate(((1, 0), (3, 1))):
            @pl.when(j == (kk + 1) * STEPS_PER_CHUNK - 1)
            def _eager(off=off, idx=idx):
                peer = lax.rem(my + off, N_DEV)
                rdma = pltpu.make_async_remote_copy(
                    src_ref=y_ref.at[:, pl.ds(peer * N_PER, N_PER)],
                    dst_ref=bf_recv.at[idx],
                    send_sem=bf_send_sems.at[idx],
                    recv_sem=bf_recv_sems.at[idx],
                    device_id=_mesh_id(peer),
                    device_id_type=pl.DeviceIdType.MESH,
                )
                rdma.start()

        @pl.when(j == G - 1)
        def _last_step():
            sc_max = jax.named_scope("max_exchange"); sc_max.__enter__()
            max_sends = []
            for d in range(1, N_DEV):
                peer = lax.rem(my + d, N_DEV)
                rdma = pltpu.make_async_remote_copy(
                    src_ref=mymax_ref,
                    dst_ref=maxes_ref.at[d - 1],
                    send_sem=max_send_sems.at[d - 1],
                    recv_sem=max_recv_sems.at[d - 1],
                    device_id=_mesh_id(peer),
                    device_id_type=pl.DeviceIdType.MESH,
                )
                rdma.start()
                max_sends.append(rdma)
            for e in range(N_DEV - 1):
                rx = pltpu.make_async_remote_copy(
                    src_ref=mymax_ref,
                    dst_ref=maxes_ref.at[e],
                    send_sem=max_send_sems.at[e],
                    recv_sem=max_recv_sems.at[e],
                    device_id=_mesh_id(my),
                    device_id_type=pl.DeviceIdType.MESH,
                )
                rx.wait_recv()
            sc_max.__exit__(None, None, None)

            gmax = jnp.maximum(jnp.max(maxes_ref[...]), mymax_ref[0, 0])
            scale = gmax / 127.0
            scale_ref[...] = jnp.full((8, 128), scale, dtype=jnp.float32)
            inv = 127.0 / gmax

            def quant_f32(yv):
                yv = jnp.maximum(yv, 0.0)
                q = jnp.clip(jnp.round(yv * inv), 0.0, 127.0)
                return q.astype(jnp.int8)

            sc_qd = jax.named_scope("quant_diag_send"); sc_qd.__enter__()
            diag = lax.rem(my + 2, N_DEV)
            send_buf[...] = quant_f32(
                y_ref[:, pl.ds(diag * N_PER, N_PER)].astype(jnp.float32))
            diag_rdma = pltpu.make_async_remote_copy(
                src_ref=send_buf,
                dst_ref=recv_ref.at[1],
                send_sem=diag_send_sem,
                recv_sem=diag_recv_sem,
                device_id=_mesh_id(diag),
                device_id_type=pl.DeviceIdType.MESH,
            )
            diag_rdma.start()
            sc_qd.__exit__(None, None, None)

            sc_qr = jax.named_scope("quant_rest"); sc_qr.__enter__()
            recv_ref[3] = quant_f32(
                y_ref[:, pl.ds(my * N_PER, N_PER)].astype(jnp.float32))

            for idx, slot in ((0, 0), (1, 2)):
                rx = pltpu.make_async_remote_copy(
                    src_ref=y_ref.at[:, pl.ds(0, N_PER)],
                    dst_ref=bf_recv.at[idx],
                    send_sem=bf_send_sems.at[idx],
                    recv_sem=bf_recv_sems.at[idx],
                    device_id=_mesh_id(my),
                    device_id_type=pl.DeviceIdType.MESH,
                )
                rx.wait_recv()
                recv_ref[slot] = quant_f32(bf_recv[idx].astype(jnp.float32))
            sc_qr.__exit__(None, None, None)

            sc_wd = jax.named_scope("wait_diag"); sc_wd.__enter__()
            rx = pltpu.make_async_remote_copy(
                src_ref=send_buf,
                dst_ref=recv_ref.at[1],
                send_sem=diag_send_sem,
                recv_sem=diag_recv_sem,
                device_id=_mesh_id(my),
                device_id_type=pl.DeviceIdType.MESH,
            )
            rx.wait_recv()
            sc_wd.__exit__(None, None, None)

            sc_dr = jax.named_scope("drain_sends"); sc_dr.__enter__()
            for s in max_sends:
                s.wait_send()
            diag_rdma.wait_send()
            for idx in range(2):
                tx = pltpu.make_async_remote_copy(
                    src_ref=y_ref.at[:, pl.ds(0, N_PER)],
                    dst_ref=bf_recv.at[idx],
                    send_sem=bf_send_sems.at[idx],
                    recv_sem=bf_recv_sems.at[idx],
                    device_id=_mesh_id(my),
                    device_id_type=pl.DeviceIdType.MESH,
                )
                tx.wait_send()
            sc_dr.__exit__(None, None, None)

    return pl.pallas_call(
        body,
        grid=(G,),
        out_shape=(
            jax.ShapeDtypeStruct((N_DEV, M_PER, N_PER), jnp.int8),
            jax.ShapeDtypeStruct((8, 128), jnp.float32),
        ),
        in_specs=[
            pl.BlockSpec((M_PER, K), lambda j: (0, 0)),
            pl.BlockSpec((K, WBLK), _w_index_map),
        ],
        out_specs=(
            pl.BlockSpec((N_DEV, M_PER, N_PER), lambda j: (0, 0, 0)),
            pl.BlockSpec((8, 128), lambda j: (0, 0)),
        ),
        scratch_shapes=[
            pltpu.VMEM((M_PER, N_TOT), jnp.bfloat16),
            pltpu.VMEM((8, 128), jnp.float32),
            pltpu.VMEM((N_DEV - 1, 8, 128), jnp.float32),
            pltpu.VMEM((M_PER, N_PER), jnp.int8),
            pltpu.VMEM((2, M_PER, N_PER), jnp.bfloat16),
            pltpu.SemaphoreType.DMA((N_DEV - 1,)),
            pltpu.SemaphoreType.DMA((N_DEV - 1,)),
            pltpu.SemaphoreType.DMA,
            pltpu.SemaphoreType.DMA,
            pltpu.SemaphoreType.DMA((2,)),
            pltpu.SemaphoreType.DMA((2,)),
        ],
        compiler_params=pltpu.CompilerParams(
            dimension_semantics=("arbitrary",),
            collective_id=0,
            vmem_limit_bytes=64 * 1024 * 1024,
            skip_device_barrier=True,
        ),
    )(x, w_mat)


def _dequant(recv_q, scale):

    def body(recv_ref, scale_ref, out_ref):
        b = pl.program_id(0)
        my = lax.axis_index("i")
        s = scale_ref[0, 0]
        k = jnp.where(b == my, 3, lax.rem(my - b + N_DEV, N_DEV) - 1)
        chunk = recv_ref[pl.ds(k, 1)]
        out_ref[...] = (chunk[0].astype(jnp.float32) * s).astype(jnp.bfloat16)

    return pl.pallas_call(
        body,
        grid=(N_DEV,),
        out_shape=jax.ShapeDtypeStruct((N_DEV * M_PER, N_PER), jnp.bfloat16),
        in_specs=[
            pl.BlockSpec((N_DEV, M_PER, N_PER), lambda b: (0, 0, 0)),
            pl.BlockSpec((8, 128), lambda b: (0, 0)),
        ],
        out_specs=pl.BlockSpec((M_PER, N_PER), lambda b: (b, 0)),
        compiler_params=pltpu.CompilerParams(
            dimension_semantics=("arbitrary",),
        ),
    )(recv_q, scale)


def kernel(x, w_mat):
    recv_q, scale = _compute_quant_a2a(_cast_bf16(x), w_mat)
    return _dequant(recv_q, scale)
